# Initial kernel scaffold; baseline (speedup 1.0000x reference)
import jax
import jax.numpy as jnp
from jax.experimental import pallas as pl


def kernel(x, edge_index, edge_attr, Wq, Wk, We, Wv, Wo, bo, bn_w, bn_b):
    raise NotImplementedError("write your pallas kernel here")


# speedup vs baseline: 8.4817x; 8.4817x over previous
import jax
import jax.numpy as jnp
from jax import lax
from jax.experimental import pallas as pl
from jax.experimental.pallas import tpu as pltpu
from jax.experimental.pallas import tpu_sc as plsc

H = 8
DH = 16
NC = 2
NS = 16
C = 64


def _qkv_body(x_ref, wq_ref, wk_ref, wv_ref, q_ref, k_ref, v_ref):
    x = x_ref[...]
    q_ref[...] = jnp.dot(x, wq_ref[...], preferred_element_type=jnp.float32)
    k_ref[...] = jnp.dot(x, wk_ref[...], preferred_element_type=jnp.float32)
    v_ref[...] = jnp.dot(x, wv_ref[...], preferred_element_type=jnp.float32)


def _ee_body(ea_ref, we_ref, ee_ref):
    ee_ref[...] = jnp.dot(ea_ref[...], we_ref[...],
                          preferred_element_type=jnp.float32)


def _out_body(wv2_ref, z2_ref, x_ref, wo_ref, bo_ref, bnw_ref, bnb_ref, o_ref):
    n = x_ref.shape[0]
    hd = H * DH
    wv = (wv2_ref[0] + wv2_ref[1])[:n]
    z = (z2_ref[0] + z2_ref[1])[:n]
    col = lax.broadcasted_iota(jnp.int32, (hd, hd), 1) // DH
    row = lax.broadcasted_iota(jnp.int32, (hd, hd), 0)
    expand = (col == row).astype(jnp.float32)
    zf = jnp.dot(z, expand, preferred_element_type=jnp.float32)
    h = wv / (zf + 1e-6)
    h = jnp.dot(h, wo_ref[...], preferred_element_type=jnp.float32)
    h = h + bo_ref[...] + x_ref[...]
    mean = jnp.mean(h, axis=0, keepdims=True)
    var = jnp.mean((h - mean) * (h - mean), axis=0, keepdims=True)
    o_ref[...] = (h - mean) * lax.rsqrt(var + 1e-5) * bnw_ref[...] + bnb_ref[...]


def _worker_geometry(e_total):
    nw = NC * NS
    t_chunks = e_total // C
    cid = lax.axis_index("c")
    sid = lax.axis_index("s")
    wid = cid * NS + sid
    nchunk = (t_chunks - wid + nw - 1) // nw
    return cid, sid, wid, nw, nchunk


def _edge_body(q_hbm, k_hbm, v_hbm, ee_hbm, src_hbm, dst_hbm,
               wv_out, sc_out, wv_sh, src_v, dst_v, kr, qr, er, srow, sem):
    n_pad = wv_sh.shape[0]
    rpt = n_pad // NS
    cid, sid, wid, nw, nchunk = _worker_geometry(src_hbm.shape[0])

    def _zrow(i, _):
        for j in range(H * DH // 16):
            kr[i, pl.ds(j * 16, 16)] = jnp.zeros((16,), jnp.float32)
            srow[i, pl.ds(j * 16, 16)] = jnp.zeros((16,), jnp.float32)
        return 0
    lax.fori_loop(0, C, _zrow, 0)

    rows0 = sid * rpt
    for i in range(rpt // C):
        pltpu.sync_copy(kr, wv_sh.at[pl.ds(rows0 + i * C, C)])
    plsc.subcore_barrier()

    iota16 = lax.iota(jnp.int32, 16)

    def chunk(ci, _):
        off = (wid + ci * nw) * C
        pltpu.sync_copy(src_hbm.at[pl.ds(off, C)], src_v)
        pltpu.sync_copy(dst_hbm.at[pl.ds(off, C)], dst_v)
        pltpu.sync_copy(ee_hbm.at[pl.ds(off, C)], er)
        ck = pltpu.async_copy(k_hbm.at[src_v], kr, sem)
        cq = pltpu.async_copy(q_hbm.at[dst_v], qr, sem)
        ck.wait()
        cq.wait()

        def group(g, _):
            rows = iota16 + g * 16
            for h in range(H):
                s = jnp.zeros((16,), jnp.float32)
                for dh in range(DH):
                    cols = jnp.full((16,), h * DH + dh, jnp.int32)
                    kv = plsc.load_gather(kr, [rows, cols])
                    qv = plsc.load_gather(qr, [rows, cols])
                    ev = plsc.load_gather(er, [rows, cols])
                    s = s + kv * qv * ev
                s = jnp.exp(jnp.clip(s * 0.25, -5.0, 5.0))
                plsc.store_scatter(srow, [rows, jnp.full((16,), h, jnp.int32)], s)
            return 0
        lax.fori_loop(0, C // 16, group, 0)

        cv = pltpu.async_copy(v_hbm.at[src_v], kr, sem)
        cv.wait()

        def group_msg(g, _):
            rows = iota16 + g * 16
            for h in range(H):
                s = plsc.load_gather(srow, [rows, jnp.full((16,), h, jnp.int32)])
                for dh in range(DH):
                    cols = jnp.full((16,), h * DH + dh, jnp.int32)
                    vv = plsc.load_gather(kr, [rows, cols])
                    plsc.store_scatter(kr, [rows, cols], vv * s)
            return 0
        lax.fori_loop(0, C // 16, group_msg, 0)

        pltpu.sync_copy(kr, wv_sh.at[dst_v], add=True)
        pltpu.sync_copy(srow, sc_out.at[pl.ds(off, C)])
        return 0
    lax.fori_loop(0, nchunk, chunk, 0)

    plsc.subcore_barrier()
    pltpu.sync_copy(wv_sh.at[pl.ds(rows0, rpt)],
                    wv_out.at[cid, pl.ds(rows0, rpt)])


def _z_body(sc_hbm, dst_hbm, z_out, z_sh, dst_v, srow):
    n_pad = z_sh.shape[0]
    rpt = n_pad // NS
    cid, sid, wid, nw, nchunk = _worker_geometry(dst_hbm.shape[0])

    def _zrow0(i, _):
        for j in range(H * DH // 16):
            srow[i, pl.ds(j * 16, 16)] = jnp.zeros((16,), jnp.float32)
        return 0
    lax.fori_loop(0, C, _zrow0, 0)

    rows0 = sid * rpt
    for i in range(rpt // C):
        pltpu.sync_copy(srow, z_sh.at[pl.ds(rows0 + i * C, C)])
    plsc.subcore_barrier()

    def chunk(ci, _):
        off = (wid + ci * nw) * C
        pltpu.sync_copy(dst_hbm.at[pl.ds(off, C)], dst_v)
        pltpu.sync_copy(sc_hbm.at[pl.ds(off, C)], srow)
        pltpu.sync_copy(srow, z_sh.at[dst_v], add=True)
        return 0
    lax.fori_loop(0, nchunk, chunk, 0)

    plsc.subcore_barrier()
    pltpu.sync_copy(z_sh.at[pl.ds(rows0, rpt)],
                    z_out.at[cid, pl.ds(rows0, rpt)])


def kernel(x, edge_index, edge_attr, Wq, Wk, We, Wv, Wo, bo, bn_w, bn_b):
    n, d = x.shape
    e = edge_index.shape[1]
    hd = H * DH
    ei = edge_index.astype(jnp.int32)
    src = ei[0]
    dst = ei[1]

    q, k, v = pl.pallas_call(
        _qkv_body,
        out_shape=[jax.ShapeDtypeStruct((n, hd), jnp.float32)] * 3,
    )(x, Wq, Wk, Wv)

    eb = 4000
    ee = pl.pallas_call(
        _ee_body,
        grid=(e // eb,),
        in_specs=[pl.BlockSpec((eb, d), lambda i: (i, 0)),
                  pl.BlockSpec((d, hd), lambda i: (0, 0))],
        out_specs=pl.BlockSpec((eb, hd), lambda i: (i, 0)),
        out_shape=jax.ShapeDtypeStruct((e, hd), jnp.float32),
    )(edge_attr, We)

    n_pad = 16 * NS * ((n + 16 * NS - 1) // (16 * NS))
    mesh = plsc.VectorSubcoreMesh(core_axis_name="c", subcore_axis_name="s",
                                  num_cores=NC, num_subcores=NS)
    wv2, scores = pl.kernel(
        _edge_body,
        out_type=[jax.ShapeDtypeStruct((NC, n_pad, hd), jnp.float32),
                  jax.ShapeDtypeStruct((e, hd), jnp.float32)],
        mesh=mesh,
        compiler_params=pltpu.CompilerParams(needs_layout_passes=False),
        scratch_types=[
            pltpu.VMEM_SHARED((n_pad, hd), jnp.float32),
            pltpu.VMEM((C,), jnp.int32),
            pltpu.VMEM((C,), jnp.int32),
            pltpu.VMEM((C, hd), jnp.float32),
            pltpu.VMEM((C, hd), jnp.float32),
            pltpu.VMEM((C, hd), jnp.float32),
            pltpu.VMEM((C, hd), jnp.float32),
            pltpu.SemaphoreType.DMA,
        ],
    )(q, k, v, ee, src, dst)

    z2 = pl.kernel(
        _z_body,
        out_type=jax.ShapeDtypeStruct((NC, n_pad, hd), jnp.float32),
        mesh=mesh,
        compiler_params=pltpu.CompilerParams(needs_layout_passes=False),
        scratch_types=[
            pltpu.VMEM_SHARED((n_pad, hd), jnp.float32),
            pltpu.VMEM((C,), jnp.int32),
            pltpu.VMEM((C, hd), jnp.float32),
        ],
    )(scores, dst)

    out = pl.pallas_call(
        _out_body,
        out_shape=jax.ShapeDtypeStruct((n, d), jnp.float32),
    )(wv2, z2, x, Wo, bo.reshape(1, d), bn_w.reshape(1, d), bn_b.reshape(1, d))
    return out


# speedup vs baseline: 8.8172x; 1.0396x over previous
import jax
import jax.numpy as jnp
from jax import lax
from jax.experimental import pallas as pl
from jax.experimental.pallas import tpu as pltpu
from jax.experimental.pallas import tpu_sc as plsc

H = 8
DH = 16
NC = 2
NS = 16
C = 64


def _qkv_body(x_ref, wq_ref, wk_ref, wv_ref, q_ref, k_ref, v_ref):
    x = x_ref[...]
    q_ref[...] = jnp.dot(x, wq_ref[...], preferred_element_type=jnp.float32)
    k_ref[...] = jnp.dot(x, wk_ref[...], preferred_element_type=jnp.float32)
    v_ref[...] = jnp.dot(x, wv_ref[...], preferred_element_type=jnp.float32)


def _ee_body(ea_ref, we_ref, ee_ref):
    ee_ref[...] = jnp.dot(ea_ref[...], we_ref[...],
                          preferred_element_type=jnp.float32)


def _out_body(wv2_ref, z2_ref, x_ref, wo_ref, bo_ref, bnw_ref, bnb_ref, o_ref):
    n = x_ref.shape[0]
    hd = H * DH
    wv = (wv2_ref[0] + wv2_ref[1])[:n]
    z = (z2_ref[0] + z2_ref[1])[:n]
    col = lax.broadcasted_iota(jnp.int32, (hd, hd), 1) // DH
    row = lax.broadcasted_iota(jnp.int32, (hd, hd), 0)
    expand = (col == row).astype(jnp.float32)
    zf = jnp.dot(z, expand, preferred_element_type=jnp.float32)
    h = wv / (zf + 1e-6)
    h = jnp.dot(h, wo_ref[...], preferred_element_type=jnp.float32)
    h = h + bo_ref[...] + x_ref[...]
    mean = jnp.mean(h, axis=0, keepdims=True)
    var = jnp.mean((h - mean) * (h - mean), axis=0, keepdims=True)
    o_ref[...] = (h - mean) * lax.rsqrt(var + 1e-5) * bnw_ref[...] + bnb_ref[...]


def _worker_geometry(e_total, c):
    nw = NC * NS
    t_chunks = e_total // c
    cid = lax.axis_index("c")
    sid = lax.axis_index("s")
    wid = cid * NS + sid
    nchunk = (t_chunks - wid + nw - 1) // nw
    return cid, sid, wid, nw, nchunk


def _edge_body(q_hbm, k_hbm, v_hbm, ee_hbm, src_hbm, dst_hbm,
               wv_out, sc_out, wv_sh, src_v, dst_v, kr, qr, er, srow, sem):
    n_pad = wv_sh.shape[0]
    rpt = n_pad // NS
    cid, sid, wid, nw, nchunk = _worker_geometry(src_hbm.shape[0], C)

    def _zrow(i, _):
        for j in range(H * DH // 16):
            kr[i, pl.ds(j * 16, 16)] = jnp.zeros((16,), jnp.float32)
            srow[i, pl.ds(j * 16, 16)] = jnp.zeros((16,), jnp.float32)
        return 0
    lax.fori_loop(0, C, _zrow, 0)

    rows0 = sid * rpt
    for i in range(rpt // C):
        pltpu.sync_copy(kr, wv_sh.at[pl.ds(rows0 + i * C, C)])
    plsc.subcore_barrier()

    iota16 = lax.iota(jnp.int32, 16)

    def chunk(ci, _):
        off = (wid + ci * nw) * C
        pltpu.sync_copy(src_hbm.at[pl.ds(off, C)], src_v)
        pltpu.sync_copy(dst_hbm.at[pl.ds(off, C)], dst_v)
        ce = pltpu.async_copy(ee_hbm.at[pl.ds(off, C)], er, sem)
        ck = pltpu.async_copy(k_hbm.at[src_v], kr, sem)
        cq = pltpu.async_copy(q_hbm.at[dst_v], qr, sem)
        ce.wait()
        ck.wait()
        cq.wait()

        def group(g, _):
            rows = iota16 + g * 16
            for h in range(H):
                s = jnp.zeros((16,), jnp.float32)
                for dh in range(DH):
                    cols = jnp.full((16,), h * DH + dh, jnp.int32)
                    kv = plsc.load_gather(kr, [rows, cols])
                    qv = plsc.load_gather(qr, [rows, cols])
                    ev = plsc.load_gather(er, [rows, cols])
                    s = s + kv * qv * ev
                s = jnp.exp(jnp.clip(s * 0.25, -5.0, 5.0))
                plsc.store_scatter(srow, [rows, jnp.full((16,), h, jnp.int32)], s)
            return 0
        lax.fori_loop(0, C // 16, group, 0)

        cv = pltpu.async_copy(v_hbm.at[src_v], kr, sem)
        cv.wait()

        def group_msg(g, _):
            rows = iota16 + g * 16
            for h in range(H):
                s = plsc.load_gather(srow, [rows, jnp.full((16,), h, jnp.int32)])
                for dh in range(DH):
                    cols = jnp.full((16,), h * DH + dh, jnp.int32)
                    vv = plsc.load_gather(kr, [rows, cols])
                    plsc.store_scatter(kr, [rows, cols], vv * s)
            return 0
        lax.fori_loop(0, C // 16, group_msg, 0)

        pltpu.sync_copy(kr, wv_sh.at[dst_v], add=True)
        pltpu.sync_copy(srow, sc_out.at[pl.ds(off, C)])
        return 0
    lax.fori_loop(0, nchunk, chunk, 0)

    plsc.subcore_barrier()
    pltpu.sync_copy(wv_sh.at[pl.ds(rows0, rpt)],
                    wv_out.at[cid, pl.ds(rows0, rpt)])


def _z_body(sc_hbm, dst_hbm, z_out, z_sh, dst_v, srow):
    n_pad = z_sh.shape[0]
    rpt = n_pad // NS
    cz = srow.shape[0]
    cid, sid, wid, nw, nchunk = _worker_geometry(dst_hbm.shape[0], cz)

    def _zrow0(i, _):
        for j in range(H * DH // 16):
            srow[i, pl.ds(j * 16, 16)] = jnp.zeros((16,), jnp.float32)
        return 0
    lax.fori_loop(0, cz, _zrow0, 0)

    rows0 = sid * rpt
    for i in range(rpt // cz):
        pltpu.sync_copy(srow, z_sh.at[pl.ds(rows0 + i * cz, cz)])
    plsc.subcore_barrier()

    def chunk(ci, _):
        off = (wid + ci * nw) * cz
        pltpu.sync_copy(dst_hbm.at[pl.ds(off, cz)], dst_v)
        pltpu.sync_copy(sc_hbm.at[pl.ds(off, cz)], srow)
        pltpu.sync_copy(srow, z_sh.at[dst_v], add=True)
        return 0
    lax.fori_loop(0, nchunk, chunk, 0)

    plsc.subcore_barrier()
    pltpu.sync_copy(z_sh.at[pl.ds(rows0, rpt)],
                    z_out.at[cid, pl.ds(rows0, rpt)])


def kernel(x, edge_index, edge_attr, Wq, Wk, We, Wv, Wo, bo, bn_w, bn_b):
    n, d = x.shape
    e = edge_index.shape[1]
    hd = H * DH
    ei = edge_index.astype(jnp.int32)
    src = ei[0]
    dst = ei[1]

    q, k, v = pl.pallas_call(
        _qkv_body,
        out_shape=[jax.ShapeDtypeStruct((n, hd), jnp.float32)] * 3,
    )(x, Wq, Wk, Wv)

    eb = 4000
    ee = pl.pallas_call(
        _ee_body,
        grid=(e // eb,),
        in_specs=[pl.BlockSpec((eb, d), lambda i: (i, 0)),
                  pl.BlockSpec((d, hd), lambda i: (0, 0))],
        out_specs=pl.BlockSpec((eb, hd), lambda i: (i, 0)),
        out_shape=jax.ShapeDtypeStruct((e, hd), jnp.float32),
    )(edge_attr, We)

    n_pad = 16 * NS * ((n + 16 * NS - 1) // (16 * NS))
    mesh = plsc.VectorSubcoreMesh(core_axis_name="c", subcore_axis_name="s",
                                  num_cores=NC, num_subcores=NS)
    wv2, scores = pl.kernel(
        _edge_body,
        out_type=[jax.ShapeDtypeStruct((NC, n_pad, hd), jnp.float32),
                  jax.ShapeDtypeStruct((e, hd), jnp.float32)],
        mesh=mesh,
        compiler_params=pltpu.CompilerParams(needs_layout_passes=False),
        scratch_types=[
            pltpu.VMEM_SHARED((n_pad, hd), jnp.float32),
            pltpu.VMEM((C,), jnp.int32),
            pltpu.VMEM((C,), jnp.int32),
            pltpu.VMEM((C, hd), jnp.float32),
            pltpu.VMEM((C, hd), jnp.float32),
            pltpu.VMEM((C, hd), jnp.float32),
            pltpu.VMEM((C, hd), jnp.float32),
            pltpu.SemaphoreType.DMA,
        ],
    )(q, k, v, ee, src, dst)

    z2 = pl.kernel(
        _z_body,
        out_type=jax.ShapeDtypeStruct((NC, n_pad, hd), jnp.float32),
        mesh=mesh,
        compiler_params=pltpu.CompilerParams(needs_layout_passes=False),
        scratch_types=[
            pltpu.VMEM_SHARED((n_pad, hd), jnp.float32),
            pltpu.VMEM((128,), jnp.int32),
            pltpu.VMEM((128, hd), jnp.float32),
        ],
    )(scores, dst)

    out = pl.pallas_call(
        _out_body,
        out_shape=jax.ShapeDtypeStruct((n, d), jnp.float32),
    )(wv2, z2, x, Wo, bo.reshape(1, d), bn_w.reshape(1, d), bn_b.reshape(1, d))
    return out
